# Initial kernel scaffold; baseline (speedup 1.0000x reference)
import jax
import jax.numpy as jnp
from jax.experimental import pallas as pl


def kernel(x, edge_index, W1, b1, W2, b2):
    raise NotImplementedError("write your pallas kernel here")


# speedup vs baseline: 140.9277x; 140.9277x over previous
import functools

import jax
import jax.numpy as jnp
from jax import lax
from jax.experimental import pallas as pl
from jax.experimental.pallas import tpu as pltpu
from jax.experimental.pallas import tpu_sc as plsc

NC = 2
NS = 16
NW = NC * NS


def _sc_edge_pass(n_pad, n_edges, window, k_planes, with_gather):
  epw = n_edges // NW
  nwin = epw // window
  chunk = n_pad // NS
  mesh = plsc.VectorSubcoreMesh(core_axis_name="c", subcore_axis_name="s")

  out_type = tuple(
      jax.ShapeDtypeStruct((NC, n_pad), jnp.float32) for _ in range(k_planes)
  )
  scratch = []
  scratch.extend(pltpu.VMEM_SHARED((n_pad,), jnp.float32) for _ in range(k_planes))
  if with_gather:
    scratch.extend(pltpu.VMEM_SHARED((n_pad,), jnp.float32) for _ in range(k_planes))
    scratch.extend(pltpu.VMEM((window,), jnp.float32) for _ in range(k_planes))
    scratch.append(pltpu.VMEM((window,), jnp.int32))
  else:
    scratch.append(pltpu.VMEM((window,), jnp.float32))
  scratch.append(pltpu.VMEM((window,), jnp.int32))

  def body(*refs):
    if with_gather:
      (src_hbm, dst_hbm, zeros_hbm, *planes_hbm) = refs[: 3 + k_planes]
      outs = refs[3 + k_planes : 3 + 2 * k_planes]
      rest = refs[3 + 2 * k_planes :]
      acc_sh = rest[:k_planes]
      tab_sh = rest[k_planes : 2 * k_planes]
      val_v = rest[2 * k_planes : 3 * k_planes]
      sidx_v = rest[3 * k_planes]
      didx_v = rest[3 * k_planes + 1]
    else:
      (dst_hbm, zeros_hbm, ones_hbm) = refs[:3]
      outs = refs[3 : 3 + k_planes]
      acc_sh = refs[3 + k_planes : 3 + 2 * k_planes]
      ones_v = refs[3 + 2 * k_planes]
      didx_v = refs[3 + 2 * k_planes + 1]

    c = lax.axis_index("c")
    s = lax.axis_index("s")
    wid = c * NS + s
    row = pl.ds(s * chunk, chunk)

    for k in range(k_planes):
      pltpu.sync_copy(zeros_hbm.at[row], acc_sh[k].at[row])
    if with_gather:
      for k in range(k_planes):
        pltpu.sync_copy(planes_hbm[k].at[row], tab_sh[k].at[row])
    else:
      pltpu.sync_copy(ones_hbm, ones_v)
    plsc.subcore_barrier()

    def step(i, carry):
      off = wid * epw + i * window
      if with_gather:
        pltpu.sync_copy(src_hbm.at[pl.ds(off, window)], sidx_v)
        pltpu.sync_copy(dst_hbm.at[pl.ds(off, window)], didx_v)
        for k in range(k_planes):
          pltpu.sync_copy(tab_sh[k].at[sidx_v], val_v[k])
        for k in range(k_planes):
          pltpu.sync_copy(val_v[k], acc_sh[k].at[didx_v], add=True)
      else:
        pltpu.sync_copy(dst_hbm.at[pl.ds(off, window)], didx_v)
        pltpu.sync_copy(ones_v, acc_sh[0].at[didx_v], add=True)
      return carry

    lax.fori_loop(0, nwin, step, 0)
    plsc.subcore_barrier()

    for k in range(k_planes):
      pltpu.sync_copy(acc_sh[k].at[row], outs[k].at[c, row])

  return pl.kernel(
      body,
      out_type=out_type,
      mesh=mesh,
      scratch_types=scratch,
  )


def _tc_node1(degp_ref, x_ref, dis_ref, y1_ref):
  deg = 1.0 + degp_ref[0] + degp_ref[1]
  dis = lax.rsqrt(deg)
  dis_ref[...] = dis
  x_r = x_ref[...].astype(jnp.bfloat16).astype(jnp.float32)
  y1_ref[...] = x_r * dis[None]


def _tc_node2(nf_in, nf_hid, accp_ref, y1_ref, dis_ref, w1_ref, b1_ref,
              w2_ref, y2_ref):
  dis = dis_ref[...]
  p = [dis * (accp_ref[0, k] + accp_ref[1, k] + y1_ref[k])
       for k in range(nf_in)]
  z = jnp.zeros_like(dis)
  for j in range(nf_hid):
    hj = jnp.full_like(dis, b1_ref[0, j])
    for k in range(nf_in):
      hj = hj + p[k] * w1_ref[k, j]
    hj = jnp.maximum(hj, 0.0)
    hj = hj.astype(jnp.bfloat16).astype(jnp.float32)
    z = z + hj * w2_ref[j, 0]
  y2_ref[...] = z * dis


def _tc_node3(accp_ref, y2_ref, dis_ref, b2_ref, out_ref):
  out_ref[...] = dis_ref[...] * (accp_ref[0] + accp_ref[1] + y2_ref[...]) \
      + b2_ref[0]


def kernel(x, edge_index, W1, b1, W2, b2):
  n = x.shape[0]
  nf_in = x.shape[1]
  nf_hid = W1.shape[1]
  n_edges = edge_index.shape[1]
  window = 2000
  n_pad = 102400
  rows = n_pad // 128

  src = edge_index[0].astype(jnp.int32)
  dst = edge_index[1].astype(jnp.int32)
  W1 = W1.astype(jnp.bfloat16).astype(jnp.float32)
  W2 = W2.astype(jnp.bfloat16).astype(jnp.float32)
  zeros_n = jnp.zeros((n_pad,), jnp.float32)
  ones_w = jnp.ones((window,), jnp.float32)

  hist = _sc_edge_pass(n_pad, n_edges, window, 1, with_gather=False)
  (degp,) = hist(dst, zeros_n, ones_w)
  degp = degp.reshape(NC, rows, 128)

  x_pad = jnp.pad(x, ((0, n_pad - n), (0, 0))).T.reshape(nf_in, rows, 128)
  dis, y1 = pl.pallas_call(
      _tc_node1,
      out_shape=(
          jax.ShapeDtypeStruct((rows, 128), jnp.float32),
          jax.ShapeDtypeStruct((nf_in, rows, 128), jnp.float32),
      ),
  )(degp, x_pad)

  gs5 = _sc_edge_pass(n_pad, n_edges, window, nf_in, with_gather=True)
  y1_planes = [y1[k].reshape(n_pad) for k in range(nf_in)]
  acc1 = gs5(src, dst, zeros_n, *y1_planes)
  acc1 = jnp.stack(acc1, axis=1).reshape(NC, nf_in, rows, 128)

  y2 = pl.pallas_call(
      functools.partial(_tc_node2, nf_in, nf_hid),
      in_specs=[
          pl.BlockSpec(memory_space=pltpu.MemorySpace.VMEM),
          pl.BlockSpec(memory_space=pltpu.MemorySpace.VMEM),
          pl.BlockSpec(memory_space=pltpu.MemorySpace.VMEM),
          pl.BlockSpec(memory_space=pltpu.SMEM),
          pl.BlockSpec(memory_space=pltpu.SMEM),
          pl.BlockSpec(memory_space=pltpu.SMEM),
      ],
      out_shape=jax.ShapeDtypeStruct((rows, 128), jnp.float32),
  )(acc1, y1, dis, W1, b1.reshape(1, nf_hid), W2)

  gs1 = _sc_edge_pass(n_pad, n_edges, window, 1, with_gather=True)
  (acc2,) = gs1(src, dst, zeros_n, y2.reshape(n_pad))
  acc2 = acc2.reshape(NC, rows, 128)

  out = pl.pallas_call(
      functools.partial(_tc_node3),
      in_specs=[
          pl.BlockSpec(memory_space=pltpu.MemorySpace.VMEM),
          pl.BlockSpec(memory_space=pltpu.MemorySpace.VMEM),
          pl.BlockSpec(memory_space=pltpu.MemorySpace.VMEM),
          pl.BlockSpec(memory_space=pltpu.SMEM),
      ],
      out_shape=jax.ShapeDtypeStruct((rows, 128), jnp.float32),
  )(acc2, y2, dis, b2)

  return out.reshape(n_pad)[:n].reshape(n, 1)


# speedup vs baseline: 187.7887x; 1.3325x over previous
import functools

import jax
import jax.numpy as jnp
from jax import lax
from jax.experimental import pallas as pl
from jax.experimental.pallas import tpu as pltpu
from jax.experimental.pallas import tpu_sc as plsc

NC = 2
NS = 16
NW = NC * NS


def _sc_hist(n_pad, n_edges, window):
  epw = n_edges // NW
  nwin = epw // window
  chunk = n_pad // NS
  mesh = plsc.VectorSubcoreMesh(core_axis_name="c", subcore_axis_name="s")

  def body(dst_hbm, zeros_hbm, ones_hbm, out_hbm, acc_sh, ones_v, didx_v):
    c = lax.axis_index("c")
    s = lax.axis_index("s")
    wid = c * NS + s
    row = pl.ds(s * chunk, chunk)

    pltpu.sync_copy(zeros_hbm.at[row], acc_sh.at[row])
    pltpu.sync_copy(ones_hbm, ones_v)
    plsc.subcore_barrier()

    def step(i, carry):
      off = wid * epw + i * window
      pltpu.sync_copy(dst_hbm.at[pl.ds(off, window)], didx_v)
      pltpu.sync_copy(ones_v, acc_sh.at[didx_v], add=True)
      return carry

    lax.fori_loop(0, nwin, step, 0)
    plsc.subcore_barrier()
    pltpu.sync_copy(acc_sh.at[row], out_hbm.at[c, row])

  return pl.kernel(
      body,
      out_type=jax.ShapeDtypeStruct((NC, n_pad), jnp.float32),
      mesh=mesh,
      compiler_params=pltpu.CompilerParams(use_tc_tiling_on_sc=False),
      scratch_types=[
          pltpu.VMEM_SHARED((n_pad,), jnp.float32),
          pltpu.VMEM((window,), jnp.float32),
          pltpu.VMEM((window,), jnp.int32),
      ],
  )


def _sc_gs_rows(n_pad, n_edges, window, fp):
  epw = n_edges // NW
  nwin = epw // window
  chunk = n_pad // NS
  mesh = plsc.VectorSubcoreMesh(core_axis_name="c", subcore_axis_name="s")

  def body(src_hbm, dst_hbm, zeros_hbm, tab_hbm, out_hbm,
           tab_sh, acc_sh, rows_v, sidx_v, didx_v):
    c = lax.axis_index("c")
    s = lax.axis_index("s")
    wid = c * NS + s
    row = pl.ds(s * chunk, chunk)

    pltpu.sync_copy(tab_hbm.at[row], tab_sh.at[row])
    pltpu.sync_copy(zeros_hbm.at[row], acc_sh.at[row])
    plsc.subcore_barrier()

    def step(i, carry):
      off = wid * epw + i * window
      pltpu.sync_copy(src_hbm.at[pl.ds(off, window)], sidx_v)
      pltpu.sync_copy(dst_hbm.at[pl.ds(off, window)], didx_v)
      pltpu.sync_copy(tab_sh.at[sidx_v], rows_v)
      pltpu.sync_copy(rows_v, acc_sh.at[didx_v], add=True)
      return carry

    lax.fori_loop(0, nwin, step, 0)
    plsc.subcore_barrier()
    pltpu.sync_copy(acc_sh.at[row], out_hbm.at[c, row])

  return pl.kernel(
      body,
      out_type=jax.ShapeDtypeStruct((NC, n_pad, fp), jnp.float32),
      mesh=mesh,
      compiler_params=pltpu.CompilerParams(use_tc_tiling_on_sc=False),
      scratch_types=[
          pltpu.VMEM_SHARED((n_pad, fp), jnp.float32),
          pltpu.VMEM_SHARED((n_pad, fp), jnp.float32),
          pltpu.VMEM((window, fp), jnp.float32),
          pltpu.VMEM((window,), jnp.int32),
          pltpu.VMEM((window,), jnp.int32),
      ],
  )


def _sc_gs_elem(n_pad, n_edges, window):
  epw = n_edges // NW
  nwin = epw // window
  chunk = n_pad // NS
  mesh = plsc.VectorSubcoreMesh(core_axis_name="c", subcore_axis_name="s")

  def body(src_hbm, dst_hbm, zeros_hbm, tab_hbm, out_hbm,
           tab_sh, acc_sh, val_v, sidx_v, didx_v):
    c = lax.axis_index("c")
    s = lax.axis_index("s")
    wid = c * NS + s
    row = pl.ds(s * chunk, chunk)

    pltpu.sync_copy(tab_hbm.at[row], tab_sh.at[row])
    pltpu.sync_copy(zeros_hbm.at[row], acc_sh.at[row])
    plsc.subcore_barrier()

    def step(i, carry):
      off = wid * epw + i * window
      pltpu.sync_copy(src_hbm.at[pl.ds(off, window)], sidx_v)
      pltpu.sync_copy(dst_hbm.at[pl.ds(off, window)], didx_v)
      pltpu.sync_copy(tab_sh.at[sidx_v], val_v)
      pltpu.sync_copy(val_v, acc_sh.at[didx_v], add=True)
      return carry

    lax.fori_loop(0, nwin, step, 0)
    plsc.subcore_barrier()
    pltpu.sync_copy(acc_sh.at[row], out_hbm.at[c, row])

  return pl.kernel(
      body,
      out_type=jax.ShapeDtypeStruct((NC, n_pad), jnp.float32),
      mesh=mesh,
      compiler_params=pltpu.CompilerParams(use_tc_tiling_on_sc=False),
      scratch_types=[
          pltpu.VMEM_SHARED((n_pad,), jnp.float32),
          pltpu.VMEM_SHARED((n_pad,), jnp.float32),
          pltpu.VMEM((window,), jnp.float32),
          pltpu.VMEM((window,), jnp.int32),
          pltpu.VMEM((window,), jnp.int32),
      ],
  )


def _tc_node1(degp_ref, x_ref, dis_ref, y1_ref):
  deg = 1.0 + degp_ref[0] + degp_ref[1]
  dis = lax.rsqrt(deg)
  dis_ref[...] = dis
  x_r = x_ref[...].astype(jnp.bfloat16).astype(jnp.float32)
  y1_ref[...] = x_r * dis[None]


def _tc_node2(nf_in, nf_hid, accp_ref, y1_ref, dis_ref, w1_ref, b1_ref,
              w2_ref, y2_ref):
  dis = dis_ref[...]
  p = [dis * (accp_ref[0, k] + accp_ref[1, k] + y1_ref[k])
       for k in range(nf_in)]
  z = jnp.zeros_like(dis)
  for j in range(nf_hid):
    hj = jnp.full_like(dis, b1_ref[0, j])
    for k in range(nf_in):
      hj = hj + p[k] * w1_ref[k, j]
    hj = jnp.maximum(hj, 0.0)
    hj = hj.astype(jnp.bfloat16).astype(jnp.float32)
    z = z + hj * w2_ref[j, 0]
  y2_ref[...] = z * dis


def _tc_node3(accp_ref, y2_ref, dis_ref, b2_ref, out_ref):
  out_ref[...] = dis_ref[...] * (accp_ref[0] + accp_ref[1] + y2_ref[...]) \
      + b2_ref[0]


def kernel(x, edge_index, W1, b1, W2, b2):
  n = x.shape[0]
  nf_in = x.shape[1]
  nf_hid = W1.shape[1]
  n_edges = edge_index.shape[1]
  window = 2000
  n_pad = 102400
  rows = n_pad // 128
  fp = 8

  src = edge_index[0].astype(jnp.int32)
  dst = edge_index[1].astype(jnp.int32)
  W1 = W1.astype(jnp.bfloat16).astype(jnp.float32)
  W2 = W2.astype(jnp.bfloat16).astype(jnp.float32)
  zeros_n = jnp.zeros((n_pad,), jnp.float32)
  ones_w = jnp.ones((window,), jnp.float32)

  degp = _sc_hist(n_pad, n_edges, window)(dst, zeros_n, ones_w)
  degp = degp.reshape(NC, rows, 128)

  x_pad = jnp.pad(x, ((0, n_pad - n), (0, 0))).T.reshape(nf_in, rows, 128)
  dis, y1 = pl.pallas_call(
      _tc_node1,
      out_shape=(
          jax.ShapeDtypeStruct((rows, 128), jnp.float32),
          jax.ShapeDtypeStruct((nf_in, rows, 128), jnp.float32),
      ),
  )(degp, x_pad)

  y1_rm = jnp.pad(y1.reshape(nf_in, n_pad).T, ((0, 0), (0, fp - nf_in)))
  zeros_nf = jnp.zeros((n_pad, fp), jnp.float32)
  acc1_rm = _sc_gs_rows(n_pad, n_edges, window, fp)(src, dst, zeros_nf, y1_rm)
  acc1 = jnp.moveaxis(acc1_rm, 2, 1)[:, :nf_in].reshape(NC, nf_in, rows, 128)

  y2 = pl.pallas_call(
      functools.partial(_tc_node2, nf_in, nf_hid),
      in_specs=[
          pl.BlockSpec(memory_space=pltpu.MemorySpace.VMEM),
          pl.BlockSpec(memory_space=pltpu.MemorySpace.VMEM),
          pl.BlockSpec(memory_space=pltpu.MemorySpace.VMEM),
          pl.BlockSpec(memory_space=pltpu.SMEM),
          pl.BlockSpec(memory_space=pltpu.SMEM),
          pl.BlockSpec(memory_space=pltpu.SMEM),
      ],
      out_shape=jax.ShapeDtypeStruct((rows, 128), jnp.float32),
  )(acc1, y1, dis, W1, b1.reshape(1, nf_hid), W2)

  acc2 = _sc_gs_elem(n_pad, n_edges, window)(
      src, dst, zeros_n, y2.reshape(n_pad))
  acc2 = acc2.reshape(NC, rows, 128)

  out = pl.pallas_call(
      _tc_node3,
      in_specs=[
          pl.BlockSpec(memory_space=pltpu.MemorySpace.VMEM),
          pl.BlockSpec(memory_space=pltpu.MemorySpace.VMEM),
          pl.BlockSpec(memory_space=pltpu.MemorySpace.VMEM),
          pl.BlockSpec(memory_space=pltpu.SMEM),
      ],
      out_shape=jax.ShapeDtypeStruct((rows, 128), jnp.float32),
  )(acc2, y2, dis, b2)

  return out.reshape(n_pad)[:n].reshape(n, 1)
